# Initial kernel scaffold; baseline (speedup 1.0000x reference)
import functools

import jax
import jax.numpy as jnp
from jax import lax
from jax.experimental import pallas as pl
from jax.experimental.pallas import tpu as pltpu
from jax.experimental.pallas import tpu_sc as plsc

_N = 10000
_NP = 10112
_PN = 10000
_E = 320000
_ETOT = _E + _N
_EP = 331776
_PT = _EP // 32
_CH = 128
_NCH = _PT // _CH
_EB = _NP
_SPR = 10368
_OC = _SPR // 16


def _sc_edge_pass(xp, scal, cvec, srcp, dstp):
    mesh = plsc.VectorSubcoreMesh(core_axis_name="c", subcore_axis_name="s")

    @functools.partial(
        pl.kernel,
        out_type=jax.ShapeDtypeStruct((2, _SPR, 128), jnp.float32),
        mesh=mesh,
        scratch_types=[
            pltpu.VMEM_SHARED((_SPR, 128), jnp.float32),
            pltpu.VMEM((_NP,), jnp.float32),
            pltpu.VMEM((_NP,), jnp.float32),
            pltpu.VMEM((_NP,), jnp.float32),
            pltpu.VMEM((_NP,), jnp.float32),
            pltpu.VMEM((80, 128), jnp.float32),
            pltpu.VMEM((80, 128), jnp.float32),
            pltpu.VMEM((_CH,), jnp.int32),
            pltpu.VMEM((_CH,), jnp.int32),
            pltpu.VMEM((_CH, 128), jnp.float32),
            pltpu.VMEM((_CH,), jnp.float32),
            pltpu.VMEM((_CH,), jnp.float32),
            pltpu.VMEM((80,), jnp.int32),
            pltpu.VMEM((80,), jnp.int32),
            pltpu.VMEM((2, 16), jnp.float32),
            pltpu.SemaphoreType.DMA,
        ],
    )
    def k(xp_hbm, scal_hbm, cvec_hbm, src_hbm, dst_hbm, out_hbm,
          sp, as0, as1, ad0, ad1, es0, es1, sidx, didx, rows, ee0, ee1,
          eidx0, eidx1, cbuf, sem):
        cid = lax.axis_index("c")
        sid = lax.axis_index("s")
        z16 = jnp.zeros((16,), jnp.float32)

        def zrow(r, carry):
            for c in range(8):
                rows[r, pl.ds(c * 16, 16)] = z16
            return carry

        lax.fori_loop(0, _CH, zrow, 0)

        def zes(r, carry):
            for c in range(8):
                es0[r, pl.ds(c * 16, 16)] = z16
                es1[r, pl.ds(c * 16, 16)] = z16
            return carry

        lax.fori_loop(0, 80, zes, 0)

        base_r = sid * _OC
        for i in range(5):
            pltpu.sync_copy(rows.at[pl.ds(0, 128)],
                            sp.at[pl.ds(base_r + i * 128, 128)])
        pltpu.sync_copy(rows.at[pl.ds(0, 8)], sp.at[pl.ds(base_r + 640, 8)])

        pltpu.sync_copy(scal_hbm.at[0], as0)
        pltpu.sync_copy(scal_hbm.at[1], as1)
        pltpu.sync_copy(scal_hbm.at[2], ad0)
        pltpu.sync_copy(scal_hbm.at[3], ad1)
        pltpu.sync_copy(cvec_hbm, cbuf)
        c0v = cbuf[0, :]
        c1v = cbuf[1, :]

        ii = lax.iota(jnp.int32, 16)
        for g in range(5):
            eidx0[pl.ds(g * 16, 16)] = ii + (_EB + g * 16)
            eidx1[pl.ds(g * 16, 16)] = ii + (_EB + 80 + g * 16)

        plsc.subcore_barrier()

        ebase = (cid * 16 + sid) * _PT

        def chunk(kk, carry):
            eoff = pl.multiple_of(ebase + kk * _CH, 8)
            pltpu.sync_copy(src_hbm.at[pl.ds(eoff, _CH)], sidx)
            pltpu.sync_copy(dst_hbm.at[pl.ds(eoff, _CH)], didx)
            cp = pltpu.async_copy(xp_hbm.at[sidx], rows, sem)
            for j in range(8):
                sv = sidx[pl.ds(16 * j, 16)]
                dv = didx[pl.ds(16 * j, 16)]
                rr = lax.shift_right_logical(dv, 7)
                cc = lax.bitwise_and(dv, 127)
                e0 = plsc.load_gather(as0, [sv]) + plsc.load_gather(ad0, [dv])
                e0 = jnp.where(e0 > 0, e0, 0.2 * e0)
                e0 = jnp.exp(e0 - c0v)
                e1 = plsc.load_gather(as1, [sv]) + plsc.load_gather(ad1, [dv])
                e1 = jnp.where(e1 > 0, e1, 0.2 * e1)
                e1 = jnp.exp(e1 - c1v)
                ee0[pl.ds(16 * j, 16)] = e0
                ee1[pl.ds(16 * j, 16)] = e1
                plsc.addupdate_scatter(es0, [rr, cc], e0)
                plsc.addupdate_scatter(es1, [rr, cc], e1)
            cp.wait()

            def scale(g, carry2):
                e0g = ee0[pl.ds(g * 16, 16)]
                e1g = ee1[pl.ds(g * 16, 16)]
                r0 = g * 16
                for t in range(16):
                    b0 = jnp.broadcast_to(e0g[t], (16,))
                    b1 = jnp.broadcast_to(e1g[t], (16,))
                    for c in range(4):
                        sl = pl.ds(c * 16, 16)
                        rows[r0 + t, sl] = rows[r0 + t, sl] * b0
                    for c in range(4, 8):
                        sl = pl.ds(c * 16, 16)
                        rows[r0 + t, sl] = rows[r0 + t, sl] * b1
                return carry2

            lax.fori_loop(0, 8, scale, 0)
            pltpu.sync_copy(rows, sp.at[didx], add=True)
            return carry

        lax.fori_loop(0, _NCH, chunk, 0)

        pltpu.sync_copy(es0, sp.at[eidx0], add=True)
        pltpu.sync_copy(es1, sp.at[eidx1], add=True)

        plsc.subcore_barrier()

        for i in range(5):
            pltpu.sync_copy(sp.at[pl.ds(base_r + i * 128, 128)],
                            out_hbm.at[cid, pl.ds(base_r + i * 128, 128)])
        pltpu.sync_copy(sp.at[pl.ds(base_r + 640, 8)],
                        out_hbm.at[cid, pl.ds(base_r + 640, 8)])

    return k(xp, scal, cvec, srcp, dstp)


def _prep_tail(h, W, A):
    xpn = jnp.dot(h, W, preferred_element_type=jnp.float32)
    scal = lax.dot_general(A, xpn, (((1,), (1,)), ((), ())),
                           preferred_element_type=jnp.float32)
    m = jnp.max(scal, axis=1, keepdims=True)
    cs = m[0:2] + m[2:4]
    cvec = jnp.broadcast_to(cs, (2, 16))
    return xpn, scal, cvec


def _tc_prep1(x_pad, W1, A1):
    def body(x_ref, w_ref, a_ref, xp_ref, scal_ref, cvec_ref):
        xpn, scal, cvec = _prep_tail(x_ref[...], w_ref[...], a_ref[...])
        xp_ref[...] = xpn
        scal_ref[...] = scal
        cvec_ref[...] = cvec

    return pl.pallas_call(
        body,
        out_shape=(
            jax.ShapeDtypeStruct((_NP, 128), jnp.float32),
            jax.ShapeDtypeStruct((4, _NP), jnp.float32),
            jax.ShapeDtypeStruct((2, 16), jnp.float32),
        ),
    )(x_pad, W1, A1)


def _finish_layer(o_ref, e_ref, b_ref, g_ref, be_ref):
    acc = o_ref[0, : _NP, :] + o_ref[1, : _NP, :]
    es0 = e_ref[0, 0] + e_ref[1, 0]
    es1 = e_ref[0, 1] + e_ref[1, 1]
    h0 = acc[:, :64] / (es0 + 1e-16)
    h1 = acc[:, 64:] / (es1 + 1e-16)
    h = jnp.concatenate([h0, h1], axis=1) + b_ref[...]
    rid = lax.broadcasted_iota(jnp.int32, (_NP, 128), 0)
    h = jnp.where(rid < _N, h, 0.0)
    mu = jnp.sum(h, axis=0, keepdims=True) / _N
    var = jnp.sum(h * h, axis=0, keepdims=True) / _N - mu * mu
    hn = g_ref[...] * (h - mu) * lax.rsqrt(var + 1e-5) + be_ref[...]
    hr = jnp.maximum(hn, 0.0)
    return jnp.where(rid < _N, hr, 0.0)


def _tc_comb(outs, esp, b, g, be, W, A):
    def body(o_ref, e_ref, b_ref, g_ref, be_ref, w_ref, a_ref,
             xp_ref, scal_ref, cvec_ref):
        hr = _finish_layer(o_ref, e_ref, b_ref, g_ref, be_ref)
        xpn, scal, cvec = _prep_tail(hr, w_ref[...], a_ref[...])
        xp_ref[...] = xpn
        scal_ref[...] = scal
        cvec_ref[...] = cvec

    return pl.pallas_call(
        body,
        out_shape=(
            jax.ShapeDtypeStruct((_NP, 128), jnp.float32),
            jax.ShapeDtypeStruct((4, _NP), jnp.float32),
            jax.ShapeDtypeStruct((2, 16), jnp.float32),
        ),
    )(outs, esp, b, g, be, W, A)


def _tc_final(outs, esp, b, g, be, batch_r, fc1_w, fc1_b, fc2_w, fc2_b):
    def body(o_ref, e_ref, b_ref, g_ref, be_ref, bt_ref,
             f1w_ref, f1b_ref, f2w_ref, f2b_ref, out_ref):
        hr = _finish_layer(o_ref, e_ref, b_ref, g_ref, be_ref)
        gid = lax.broadcasted_iota(jnp.int32, (64, _NP), 0)
        ohT = (gid == bt_ref[...]).astype(jnp.float32)
        pooled_s = lax.dot_general(ohT, hr, (((1,), (0,)), ((), ())),
                                   preferred_element_type=jnp.float32)
        ones = jnp.ones((_NP, 1), jnp.float32)
        counts = lax.dot_general(ohT, ones, (((1,), (0,)), ((), ())),
                                 preferred_element_type=jnp.float32)
        pooled = pooled_s / jnp.maximum(counts, 1.0)
        hm = jnp.maximum(
            jnp.dot(pooled, f1w_ref[...],
                    preferred_element_type=jnp.float32) + f1b_ref[...], 0.0)
        out_ref[...] = jnp.dot(hm, f2w_ref[...],
                               preferred_element_type=jnp.float32) + f2b_ref[...]

    return pl.pallas_call(
        body,
        out_shape=jax.ShapeDtypeStruct((64, 128), jnp.float32),
    )(outs, esp, b, g, be, batch_r, fc1_w, fc1_b, fc2_w, fc2_b)


def _esum_view(o):
    return (o[:, _EB:_EB + 160, :]
            .reshape(2, 2, 10240)[:, :, :_NP]
            .reshape(2, 2, _NP, 1))


def kernel(x, edge_index, batch,
           W1, a_src1, a_dst1, b1, g1, be1,
           W2, a_src2, a_dst2, b2, g2, be2,
           W3, a_src3, a_dst3, b3, g3, be3,
           fc1_w, fc1_b, fc2_w, fc2_b):
    f32 = jnp.float32
    i32 = jnp.int32
    x_pad = jnp.concatenate([x, jnp.zeros((_NP - _N, 128), f32)], axis=0)
    loop = jnp.arange(_N, dtype=i32)
    padv = jnp.full((_EP - _ETOT,), _PN, i32)
    src = jnp.concatenate([edge_index[0].astype(i32), loop, padv])
    dst = jnp.concatenate([edge_index[1].astype(i32), loop, padv])
    batch_r = jnp.concatenate(
        [batch.astype(i32), jnp.full((_NP - _N,), 64, i32)]).reshape(1, _NP)

    def mkA(a_s, a_d):
        z = jnp.zeros((64,), f32)
        return jnp.stack([
            jnp.concatenate([a_s[0], z]),
            jnp.concatenate([z, a_s[1]]),
            jnp.concatenate([a_d[0], z]),
            jnp.concatenate([z, a_d[1]]),
        ])

    A1, A2, A3 = mkA(a_src1, a_dst1), mkA(a_src2, a_dst2), mkA(a_src3, a_dst3)
    r = lambda v: v.reshape(1, 128)

    xp, scal, cvec = _tc_prep1(x_pad, W1, A1)
    o1 = _sc_edge_pass(xp, scal, cvec, src, dst)
    xp, scal, cvec = _tc_comb(o1, _esum_view(o1), r(b1), r(g1), r(be1), W2, A2)
    o2 = _sc_edge_pass(xp, scal, cvec, src, dst)
    xp, scal, cvec = _tc_comb(o2, _esum_view(o2), r(b2), r(g2), r(be2), W3, A3)
    o3 = _sc_edge_pass(xp, scal, cvec, src, dst)
    return _tc_final(o3, _esum_view(o3), r(b3), r(g3), r(be3), batch_r,
                     fc1_w, r(fc1_b), fc2_w, r(fc2_b))


# speedup vs baseline: 57.9958x; 57.9958x over previous
import functools

import jax
import jax.numpy as jnp
from jax import lax
from jax.experimental import pallas as pl
from jax.experimental.pallas import tpu as pltpu
from jax.experimental.pallas import tpu_sc as plsc

_N = 10000
_NP = 10112
_PN = 10000
_E = 320000
_ETOT = _E + _N
_EP = 331776
_PT = _EP // 32
_CH = 128
_NCH = _PT // _CH
_NR = _NP // 16

_SC_PARAMS = pltpu.CompilerParams(needs_layout_passes=False)


def _sc_attn_pass(scal, cvec, srcp, dstp):
    mesh = plsc.VectorSubcoreMesh(core_axis_name="c", subcore_axis_name="s")

    @functools.partial(
        pl.kernel,
        out_type=(
            jax.ShapeDtypeStruct((2, _EP), jnp.float32),
            jax.ShapeDtypeStruct((2, 160, 128), jnp.float32),
        ),
        mesh=mesh,
        compiler_params=_SC_PARAMS,
        scratch_types=[
            pltpu.VMEM_SHARED((160, 128), jnp.float32),
            pltpu.VMEM((_NP,), jnp.float32),
            pltpu.VMEM((_NP,), jnp.float32),
            pltpu.VMEM((_NP,), jnp.float32),
            pltpu.VMEM((_NP,), jnp.float32),
            pltpu.VMEM((80, 128), jnp.float32),
            pltpu.VMEM((80, 128), jnp.float32),
            pltpu.VMEM((_CH,), jnp.int32),
            pltpu.VMEM((_CH,), jnp.int32),
            pltpu.VMEM((_CH,), jnp.float32),
            pltpu.VMEM((_CH,), jnp.float32),
            pltpu.VMEM((80,), jnp.int32),
            pltpu.VMEM((80,), jnp.int32),
            pltpu.VMEM((2, 16), jnp.float32),
        ],
    )
    def k(scal_hbm, cvec_hbm, src_hbm, dst_hbm, ee_hbm, esum_hbm,
          sp, as0, as1, ad0, ad1, es0, es1, sidx, didx, ee0, ee1,
          eidx0, eidx1, cbuf):
        cid = lax.axis_index("c")
        sid = lax.axis_index("s")
        z16 = jnp.zeros((16,), jnp.float32)

        def zes(r, carry):
            for c in range(8):
                es0[r, pl.ds(c * 16, 16)] = z16
                es1[r, pl.ds(c * 16, 16)] = z16
            return carry

        lax.fori_loop(0, 80, zes, 0)

        @pl.when(sid < 10)
        def _():
            pltpu.sync_copy(es0.at[pl.ds(0, 16)],
                            sp.at[pl.ds(sid * 16, 16)])

        pltpu.sync_copy(scal_hbm.at[0], as0)
        pltpu.sync_copy(scal_hbm.at[1], as1)
        pltpu.sync_copy(scal_hbm.at[2], ad0)
        pltpu.sync_copy(scal_hbm.at[3], ad1)
        pltpu.sync_copy(cvec_hbm, cbuf)
        c0v = cbuf[0, :]
        c1v = cbuf[1, :]

        ii = lax.iota(jnp.int32, 16)
        for g in range(5):
            eidx0[pl.ds(g * 16, 16)] = ii + g * 16
            eidx1[pl.ds(g * 16, 16)] = ii + (80 + g * 16)

        plsc.subcore_barrier()

        ebase = (cid * 16 + sid) * _PT

        def chunk(kk, carry):
            eoff = pl.multiple_of(ebase + kk * _CH, 8)
            pltpu.sync_copy(src_hbm.at[pl.ds(eoff, _CH)], sidx)
            pltpu.sync_copy(dst_hbm.at[pl.ds(eoff, _CH)], didx)
            for j in range(8):
                sv = sidx[pl.ds(16 * j, 16)]
                dv = didx[pl.ds(16 * j, 16)]
                rr = lax.shift_right_logical(dv, 7)
                cc = lax.bitwise_and(dv, 127)
                e0 = plsc.load_gather(as0, [sv]) + plsc.load_gather(ad0, [dv])
                e0 = jnp.where(e0 > 0, e0, 0.2 * e0)
                e0 = jnp.exp(e0 - c0v)
                e1 = plsc.load_gather(as1, [sv]) + plsc.load_gather(ad1, [dv])
                e1 = jnp.where(e1 > 0, e1, 0.2 * e1)
                e1 = jnp.exp(e1 - c1v)
                ee0[pl.ds(16 * j, 16)] = e0
                ee1[pl.ds(16 * j, 16)] = e1
                plsc.addupdate_scatter(es0, [rr, cc], e0)
                plsc.addupdate_scatter(es1, [rr, cc], e1)
            pltpu.sync_copy(ee0, ee_hbm.at[0, pl.ds(eoff, _CH)])
            pltpu.sync_copy(ee1, ee_hbm.at[1, pl.ds(eoff, _CH)])
            return carry

        lax.fori_loop(0, _NCH, chunk, 0)

        pltpu.sync_copy(es0, sp.at[eidx0], add=True)
        pltpu.sync_copy(es1, sp.at[eidx1], add=True)

        plsc.subcore_barrier()

        @pl.when(sid < 10)
        def _():
            pltpu.sync_copy(sp.at[pl.ds(sid * 16, 16)],
                            esum_hbm.at[cid, pl.ds(sid * 16, 16)])

    return k(scal, cvec, srcp, dstp)


def _sc_agg_pass(xp, ee, srcp, dstp):
    mesh = plsc.VectorSubcoreMesh(core_axis_name="c", subcore_axis_name="s")

    @functools.partial(
        pl.kernel,
        out_type=jax.ShapeDtypeStruct((2, _NP, 128), jnp.float32),
        mesh=mesh,
        compiler_params=_SC_PARAMS,
        scratch_types=[
            pltpu.VMEM_SHARED((_NP, 128), jnp.float32),
            pltpu.VMEM((_CH,), jnp.int32),
            pltpu.VMEM((_CH,), jnp.int32),
            pltpu.VMEM((_CH, 128), jnp.float32),
            pltpu.VMEM((_CH,), jnp.float32),
            pltpu.VMEM((_CH,), jnp.float32),
            pltpu.SemaphoreType.DMA,
        ],
    )
    def k(xp_hbm, ee_hbm, src_hbm, dst_hbm, out_hbm,
          sp, sidx, didx, rows, ee0, ee1, sem):
        cid = lax.axis_index("c")
        sid = lax.axis_index("s")
        z16 = jnp.zeros((16,), jnp.float32)

        def zrow(r, carry):
            for c in range(8):
                rows[r, pl.ds(c * 16, 16)] = z16
            return carry

        lax.fori_loop(0, _CH, zrow, 0)

        base_r = sid * _NR
        for i in range(4):
            pltpu.sync_copy(rows.at[pl.ds(0, 128)],
                            sp.at[pl.ds(base_r + i * 128, 128)])
        pltpu.sync_copy(rows.at[pl.ds(0, 120)],
                        sp.at[pl.ds(base_r + 512, 120)])

        plsc.subcore_barrier()

        ebase = (cid * 16 + sid) * _PT

        def chunk(kk, carry):
            eoff = pl.multiple_of(ebase + kk * _CH, 8)
            pltpu.sync_copy(src_hbm.at[pl.ds(eoff, _CH)], sidx)
            cp = pltpu.async_copy(xp_hbm.at[sidx], rows, sem)
            pltpu.sync_copy(dst_hbm.at[pl.ds(eoff, _CH)], didx)
            pltpu.sync_copy(ee_hbm.at[0, pl.ds(eoff, _CH)], ee0)
            pltpu.sync_copy(ee_hbm.at[1, pl.ds(eoff, _CH)], ee1)
            cp.wait()

            def scale(g, carry2):
                e0g = ee0[pl.ds(g * 16, 16)]
                e1g = ee1[pl.ds(g * 16, 16)]
                r0 = g * 16
                for t in range(16):
                    b0 = jnp.broadcast_to(e0g[t], (16,))
                    b1 = jnp.broadcast_to(e1g[t], (16,))
                    for c in range(4):
                        sl = pl.ds(c * 16, 16)
                        rows[r0 + t, sl] = rows[r0 + t, sl] * b0
                    for c in range(4, 8):
                        sl = pl.ds(c * 16, 16)
                        rows[r0 + t, sl] = rows[r0 + t, sl] * b1
                return carry2

            lax.fori_loop(0, 8, scale, 0)
            pltpu.sync_copy(rows, sp.at[didx], add=True)
            return carry

        lax.fori_loop(0, _NCH, chunk, 0)

        plsc.subcore_barrier()

        for i in range(4):
            pltpu.sync_copy(sp.at[pl.ds(base_r + i * 128, 128)],
                            out_hbm.at[cid, pl.ds(base_r + i * 128, 128)])
        pltpu.sync_copy(sp.at[pl.ds(base_r + 512, 120)],
                        out_hbm.at[cid, pl.ds(base_r + 512, 120)])

    return k(xp, ee, srcp, dstp)


def _sc_edge_pass(xp, scal, cvec, srcp, dstp):
    ee, esum = _sc_attn_pass(scal, cvec, srcp, dstp)
    acc = _sc_agg_pass(xp, ee, srcp, dstp)
    return acc, esum


def _prep_tail(h, W, A):
    xpn = jnp.dot(h, W, preferred_element_type=jnp.float32)
    scal = lax.dot_general(A, xpn, (((1,), (1,)), ((), ())),
                           preferred_element_type=jnp.float32)
    m = jnp.max(scal, axis=1, keepdims=True)
    cs = m[0:2] + m[2:4]
    cvec = jnp.broadcast_to(cs, (2, 16))
    return xpn, scal, cvec


def _tc_prep1(x_pad, W1, A1):
    def body(x_ref, w_ref, a_ref, xp_ref, scal_ref, cvec_ref):
        xpn, scal, cvec = _prep_tail(x_ref[...], w_ref[...], a_ref[...])
        xp_ref[...] = xpn
        scal_ref[...] = scal
        cvec_ref[...] = cvec

    return pl.pallas_call(
        body,
        out_shape=(
            jax.ShapeDtypeStruct((_NP, 128), jnp.float32),
            jax.ShapeDtypeStruct((4, _NP), jnp.float32),
            jax.ShapeDtypeStruct((2, 16), jnp.float32),
        ),
    )(x_pad, W1, A1)


def _finish_layer(o_ref, e_ref, b_ref, g_ref, be_ref):
    acc = o_ref[0] + o_ref[1]
    es0 = e_ref[0, 0] + e_ref[1, 0]
    es1 = e_ref[0, 1] + e_ref[1, 1]
    h0 = acc[:, :64] / (es0 + 1e-16)
    h1 = acc[:, 64:] / (es1 + 1e-16)
    h = jnp.concatenate([h0, h1], axis=1) + b_ref[...]
    rid = lax.broadcasted_iota(jnp.int32, (_NP, 128), 0)
    h = jnp.where(rid < _N, h, 0.0)
    mu = jnp.sum(h, axis=0, keepdims=True) / _N
    var = jnp.sum(h * h, axis=0, keepdims=True) / _N - mu * mu
    hn = g_ref[...] * (h - mu) * lax.rsqrt(var + 1e-5) + be_ref[...]
    hr = jnp.maximum(hn, 0.0)
    return jnp.where(rid < _N, hr, 0.0)


def _tc_comb(outs, esp, b, g, be, W, A):
    def body(o_ref, e_ref, b_ref, g_ref, be_ref, w_ref, a_ref,
             xp_ref, scal_ref, cvec_ref):
        hr = _finish_layer(o_ref, e_ref, b_ref, g_ref, be_ref)
        xpn, scal, cvec = _prep_tail(hr, w_ref[...], a_ref[...])
        xp_ref[...] = xpn
        scal_ref[...] = scal
        cvec_ref[...] = cvec

    return pl.pallas_call(
        body,
        out_shape=(
            jax.ShapeDtypeStruct((_NP, 128), jnp.float32),
            jax.ShapeDtypeStruct((4, _NP), jnp.float32),
            jax.ShapeDtypeStruct((2, 16), jnp.float32),
        ),
    )(outs, esp, b, g, be, W, A)


def _tc_final(outs, esp, b, g, be, batch_r, fc1_w, fc1_b, fc2_w, fc2_b):
    def body(o_ref, e_ref, b_ref, g_ref, be_ref, bt_ref,
             f1w_ref, f1b_ref, f2w_ref, f2b_ref, out_ref):
        hr = _finish_layer(o_ref, e_ref, b_ref, g_ref, be_ref)
        gid = lax.broadcasted_iota(jnp.int32, (64, _NP), 0)
        ohT = (gid == bt_ref[...]).astype(jnp.float32)
        pooled_s = lax.dot_general(ohT, hr, (((1,), (0,)), ((), ())),
                                   preferred_element_type=jnp.float32)
        ones = jnp.ones((_NP, 1), jnp.float32)
        counts = lax.dot_general(ohT, ones, (((1,), (0,)), ((), ())),
                                 preferred_element_type=jnp.float32)
        pooled = pooled_s / jnp.maximum(counts, 1.0)
        hm = jnp.maximum(
            jnp.dot(pooled, f1w_ref[...],
                    preferred_element_type=jnp.float32) + f1b_ref[...], 0.0)
        out_ref[...] = jnp.dot(hm, f2w_ref[...],
                               preferred_element_type=jnp.float32) + f2b_ref[...]

    return pl.pallas_call(
        body,
        out_shape=jax.ShapeDtypeStruct((64, 128), jnp.float32),
    )(outs, esp, b, g, be, batch_r, fc1_w, fc1_b, fc2_w, fc2_b)


def _esum_view(esum):
    return (esum.reshape(2, 2, 10240)[:, :, :_NP]
            .reshape(2, 2, _NP, 1))


def kernel(x, edge_index, batch,
           W1, a_src1, a_dst1, b1, g1, be1,
           W2, a_src2, a_dst2, b2, g2, be2,
           W3, a_src3, a_dst3, b3, g3, be3,
           fc1_w, fc1_b, fc2_w, fc2_b):
    f32 = jnp.float32
    i32 = jnp.int32
    x_pad = jnp.concatenate([x, jnp.zeros((_NP - _N, 128), f32)], axis=0)
    loop = jnp.arange(_N, dtype=i32)
    padv = jnp.full((_EP - _ETOT,), _PN, i32)
    src = jnp.concatenate([edge_index[0].astype(i32), loop, padv])
    dst = jnp.concatenate([edge_index[1].astype(i32), loop, padv])
    batch_r = jnp.concatenate(
        [batch.astype(i32), jnp.full((_NP - _N,), 64, i32)]).reshape(1, _NP)

    def mkA(a_s, a_d):
        z = jnp.zeros((64,), f32)
        return jnp.stack([
            jnp.concatenate([a_s[0], z]),
            jnp.concatenate([z, a_s[1]]),
            jnp.concatenate([a_d[0], z]),
            jnp.concatenate([z, a_d[1]]),
        ])

    A1, A2, A3 = mkA(a_src1, a_dst1), mkA(a_src2, a_dst2), mkA(a_src3, a_dst3)
    r = lambda v: v.reshape(1, 128)

    xp, scal, cvec = _tc_prep1(x_pad, W1, A1)
    o1, s1 = _sc_edge_pass(xp, scal, cvec, src, dst)
    xp, scal, cvec = _tc_comb(o1, _esum_view(s1), r(b1), r(g1), r(be1), W2, A2)
    o2, s2 = _sc_edge_pass(xp, scal, cvec, src, dst)
    xp, scal, cvec = _tc_comb(o2, _esum_view(s2), r(b2), r(g2), r(be2), W3, A3)
    o3, s3 = _sc_edge_pass(xp, scal, cvec, src, dst)
    return _tc_final(o3, _esum_view(s3), r(b3), r(g3), r(be3), batch_r,
                     fc1_w, r(fc1_b), fc2_w, r(fc2_b))


# speedup vs baseline: 77.2418x; 1.3319x over previous
import functools

import jax
import jax.numpy as jnp
from jax import lax
from jax.experimental import pallas as pl
from jax.experimental.pallas import tpu as pltpu
from jax.experimental.pallas import tpu_sc as plsc

_N = 10000
_NP = 10112
_PN = 10000
_E = 320000
_ETOT = _E + _N
_EP = 331776
_PT = _EP // 32
_CH = 128
_NCH = _PT // _CH
_CKS = _EP // _CH
_BT = 9
_NR = _NP // 16

_SC_PARAMS = pltpu.CompilerParams(needs_layout_passes=False)


def _sc_attn_pass(scal, cvec, edges):
    mesh = plsc.VectorSubcoreMesh(core_axis_name="c", subcore_axis_name="s")

    @functools.partial(
        pl.kernel,
        out_type=(
            jax.ShapeDtypeStruct((_CKS, 2, 128), jnp.float32),
            jax.ShapeDtypeStruct((2, 160, 128), jnp.float32),
        ),
        mesh=mesh,
        compiler_params=_SC_PARAMS,
        scratch_types=[
            pltpu.VMEM_SHARED((160, 128), jnp.float32),
            pltpu.VMEM((_NP,), jnp.float32),
            pltpu.VMEM((_NP,), jnp.float32),
            pltpu.VMEM((_NP,), jnp.float32),
            pltpu.VMEM((_NP,), jnp.float32),
            pltpu.VMEM((80, 128), jnp.float32),
            pltpu.VMEM((80, 128), jnp.float32),
            pltpu.VMEM((_BT, 2, 128), jnp.int32),
            pltpu.VMEM((_BT, 2, 128), jnp.float32),
            pltpu.VMEM((80,), jnp.int32),
            pltpu.VMEM((80,), jnp.int32),
            pltpu.VMEM((2, 16), jnp.float32),
        ],
    )
    def k(scal_hbm, cvec_hbm, ed_hbm, ee_hbm, esum_hbm,
          sp, as0, as1, ad0, ad1, es0, es1, edb, eeb, eidx0, eidx1, cbuf):
        cid = lax.axis_index("c")
        sid = lax.axis_index("s")
        z16 = jnp.zeros((16,), jnp.float32)

        def zes(r, carry):
            for c in range(8):
                es0[r, pl.ds(c * 16, 16)] = z16
                es1[r, pl.ds(c * 16, 16)] = z16
            return carry

        lax.fori_loop(0, 80, zes, 0)

        @pl.when(sid < 10)
        def _():
            pltpu.sync_copy(es0.at[pl.ds(0, 16)],
                            sp.at[pl.ds(sid * 16, 16)])

        pltpu.sync_copy(scal_hbm.at[0], as0)
        pltpu.sync_copy(scal_hbm.at[1], as1)
        pltpu.sync_copy(scal_hbm.at[2], ad0)
        pltpu.sync_copy(scal_hbm.at[3], ad1)
        pltpu.sync_copy(cvec_hbm, cbuf)
        c0v = cbuf[0, :]
        c1v = cbuf[1, :]

        ii = lax.iota(jnp.int32, 16)
        for g in range(5):
            eidx0[pl.ds(g * 16, 16)] = ii + g * 16
            eidx1[pl.ds(g * 16, 16)] = ii + (80 + g * 16)

        plsc.subcore_barrier()

        ckb = (cid * 16 + sid) * _NCH

        def batch(tt, carry):
            ck = ckb + tt * _BT
            pltpu.sync_copy(ed_hbm.at[pl.ds(ck, _BT)], edb)
            for u in range(_BT):
                for j in range(8):
                    sl = pl.ds(16 * j, 16)
                    sv = edb[u, 0, sl]
                    dv = edb[u, 1, sl]
                    rr = lax.shift_right_logical(dv, 7)
                    cc = lax.bitwise_and(dv, 127)
                    e0 = (plsc.load_gather(as0, [sv])
                          + plsc.load_gather(ad0, [dv]))
                    e0 = jnp.where(e0 > 0, e0, 0.2 * e0)
                    e0 = jnp.exp(e0 - c0v)
                    e1 = (plsc.load_gather(as1, [sv])
                          + plsc.load_gather(ad1, [dv]))
                    e1 = jnp.where(e1 > 0, e1, 0.2 * e1)
                    e1 = jnp.exp(e1 - c1v)
                    eeb[u, 0, sl] = e0
                    eeb[u, 1, sl] = e1
                    plsc.addupdate_scatter(es0, [rr, cc], e0)
                    plsc.addupdate_scatter(es1, [rr, cc], e1)
            pltpu.sync_copy(eeb, ee_hbm.at[pl.ds(ck, _BT)])
            return carry

        lax.fori_loop(0, _NCH // _BT, batch, 0)

        pltpu.sync_copy(es0, sp.at[eidx0], add=True)
        pltpu.sync_copy(es1, sp.at[eidx1], add=True)

        plsc.subcore_barrier()

        @pl.when(sid < 10)
        def _():
            pltpu.sync_copy(sp.at[pl.ds(sid * 16, 16)],
                            esum_hbm.at[cid, pl.ds(sid * 16, 16)])

    return k(scal, cvec, edges)


def _sc_agg_pass(xp, ee, edges):
    mesh = plsc.VectorSubcoreMesh(core_axis_name="c", subcore_axis_name="s")

    @functools.partial(
        pl.kernel,
        out_type=jax.ShapeDtypeStruct((2, _NP, 128), jnp.float32),
        mesh=mesh,
        compiler_params=_SC_PARAMS,
        scratch_types=[
            pltpu.VMEM_SHARED((_NP, 128), jnp.float32),
            pltpu.VMEM((2, 2, 128), jnp.int32),
            pltpu.VMEM((2, 2, 128), jnp.float32),
            pltpu.VMEM((_CH, 128), jnp.float32),
            pltpu.VMEM((_CH, 128), jnp.float32),
            pltpu.SemaphoreType.DMA,
            pltpu.SemaphoreType.DMA,
            pltpu.SemaphoreType.DMA,
            pltpu.SemaphoreType.DMA,
        ],
    )
    def k(xp_hbm, ee_hbm, ed_hbm, out_hbm,
          sp, edAB, eeAB, rowsA, rowsB, sga, sgb, ssa, ssb):
        cid = lax.axis_index("c")
        sid = lax.axis_index("s")
        z16 = jnp.zeros((16,), jnp.float32)

        def zrow(r, carry):
            for c in range(8):
                rowsA[r, pl.ds(c * 16, 16)] = z16
            return carry

        lax.fori_loop(0, _CH, zrow, 0)

        base_r = sid * _NR
        for i in range(4):
            pltpu.sync_copy(rowsA.at[pl.ds(0, 128)],
                            sp.at[pl.ds(base_r + i * 128, 128)])
        pltpu.sync_copy(rowsA.at[pl.ds(0, 120)],
                        sp.at[pl.ds(base_r + 512, 120)])

        plsc.subcore_barrier()

        ckb = (cid * 16 + sid) * _NCH

        def scale(rows, a):
            def body(g, carry):
                e0g = eeAB[a, 0, pl.ds(g * 16, 16)]
                e1g = eeAB[a, 1, pl.ds(g * 16, 16)]
                r0 = g * 16
                for t in range(16):
                    b0 = jnp.broadcast_to(e0g[t], (16,))
                    b1 = jnp.broadcast_to(e1g[t], (16,))
                    for c in range(4):
                        sl = pl.ds(c * 16, 16)
                        rows[r0 + t, sl] = rows[r0 + t, sl] * b0
                    for c in range(4, 8):
                        sl = pl.ds(c * 16, 16)
                        rows[r0 + t, sl] = rows[r0 + t, sl] * b1
                return carry

            lax.fori_loop(0, 8, body, 0)

        def pair(i, carry):
            ck0 = ckb + 2 * i
            pltpu.sync_copy(ed_hbm.at[pl.ds(ck0, 2)], edAB)
            pltpu.sync_copy(ee_hbm.at[pl.ds(ck0, 2)], eeAB)
            gA = pltpu.async_copy(xp_hbm.at[edAB.at[0, 0]], rowsA, sga)
            gB = pltpu.async_copy(xp_hbm.at[edAB.at[1, 0]], rowsB, sgb)
            gA.wait()
            scale(rowsA, 0)
            sA = pltpu.async_copy(rowsA, sp.at[edAB.at[0, 1]], ssa, add=True)
            gB.wait()
            scale(rowsB, 1)
            sB = pltpu.async_copy(rowsB, sp.at[edAB.at[1, 1]], ssb, add=True)
            sA.wait()
            sB.wait()
            return carry

        lax.fori_loop(0, _NCH // 2, pair, 0)

        ckl = ckb + _NCH - 1
        pltpu.sync_copy(ed_hbm.at[pl.ds(ckl, 1)], edAB.at[pl.ds(0, 1)])
        pltpu.sync_copy(ee_hbm.at[pl.ds(ckl, 1)], eeAB.at[pl.ds(0, 1)])
        pltpu.async_copy(xp_hbm.at[edAB.at[0, 0]], rowsA, sga).wait()
        scale(rowsA, 0)
        pltpu.sync_copy(rowsA, sp.at[edAB.at[0, 1]], add=True)

        plsc.subcore_barrier()

        for i in range(4):
            pltpu.sync_copy(sp.at[pl.ds(base_r + i * 128, 128)],
                            out_hbm.at[cid, pl.ds(base_r + i * 128, 128)])
        pltpu.sync_copy(sp.at[pl.ds(base_r + 512, 120)],
                        out_hbm.at[cid, pl.ds(base_r + 512, 120)])

    return k(xp, ee, edges)


def _sc_edge_pass(xp, scal, cvec, edges):
    ee, esum = _sc_attn_pass(scal, cvec, edges)
    acc = _sc_agg_pass(xp, ee, edges)
    return acc, esum


def _prep_tail(h, W, A):
    xpn = jnp.dot(h, W, preferred_element_type=jnp.float32)
    scal = lax.dot_general(A, xpn, (((1,), (1,)), ((), ())),
                           preferred_element_type=jnp.float32)
    m = jnp.max(scal, axis=1, keepdims=True)
    cs = m[0:2] + m[2:4]
    cvec = jnp.broadcast_to(cs, (2, 16))
    return xpn, scal, cvec


def _tc_prep1(x_pad, W1, A1):
    def body(x_ref, w_ref, a_ref, xp_ref, scal_ref, cvec_ref):
        xpn, scal, cvec = _prep_tail(x_ref[...], w_ref[...], a_ref[...])
        xp_ref[...] = xpn
        scal_ref[...] = scal
        cvec_ref[...] = cvec

    return pl.pallas_call(
        body,
        out_shape=(
            jax.ShapeDtypeStruct((_NP, 128), jnp.float32),
            jax.ShapeDtypeStruct((4, _NP), jnp.float32),
            jax.ShapeDtypeStruct((2, 16), jnp.float32),
        ),
    )(x_pad, W1, A1)


def _finish_layer(o_ref, e_ref, b_ref, g_ref, be_ref):
    acc = o_ref[0] + o_ref[1]
    es0 = e_ref[0, 0] + e_ref[1, 0]
    es1 = e_ref[0, 1] + e_ref[1, 1]
    h0 = acc[:, :64] / (es0 + 1e-16)
    h1 = acc[:, 64:] / (es1 + 1e-16)
    h = jnp.concatenate([h0, h1], axis=1) + b_ref[...]
    rid = lax.broadcasted_iota(jnp.int32, (_NP, 128), 0)
    h = jnp.where(rid < _N, h, 0.0)
    mu = jnp.sum(h, axis=0, keepdims=True) / _N
    var = jnp.sum(h * h, axis=0, keepdims=True) / _N - mu * mu
    hn = g_ref[...] * (h - mu) * lax.rsqrt(var + 1e-5) + be_ref[...]
    hr = jnp.maximum(hn, 0.0)
    return jnp.where(rid < _N, hr, 0.0)


def _tc_comb(outs, esp, b, g, be, W, A):
    def body(o_ref, e_ref, b_ref, g_ref, be_ref, w_ref, a_ref,
             xp_ref, scal_ref, cvec_ref):
        hr = _finish_layer(o_ref, e_ref, b_ref, g_ref, be_ref)
        xpn, scal, cvec = _prep_tail(hr, w_ref[...], a_ref[...])
        xp_ref[...] = xpn
        scal_ref[...] = scal
        cvec_ref[...] = cvec

    return pl.pallas_call(
        body,
        out_shape=(
            jax.ShapeDtypeStruct((_NP, 128), jnp.float32),
            jax.ShapeDtypeStruct((4, _NP), jnp.float32),
            jax.ShapeDtypeStruct((2, 16), jnp.float32),
        ),
    )(outs, esp, b, g, be, W, A)


def _tc_final(outs, esp, b, g, be, batch_r, fc1_w, fc1_b, fc2_w, fc2_b):
    def body(o_ref, e_ref, b_ref, g_ref, be_ref, bt_ref,
             f1w_ref, f1b_ref, f2w_ref, f2b_ref, out_ref):
        hr = _finish_layer(o_ref, e_ref, b_ref, g_ref, be_ref)
        gid = lax.broadcasted_iota(jnp.int32, (64, _NP), 0)
        ohT = (gid == bt_ref[...]).astype(jnp.float32)
        pooled_s = lax.dot_general(ohT, hr, (((1,), (0,)), ((), ())),
                                   preferred_element_type=jnp.float32)
        ones = jnp.ones((_NP, 1), jnp.float32)
        counts = lax.dot_general(ohT, ones, (((1,), (0,)), ((), ())),
                                 preferred_element_type=jnp.float32)
        pooled = pooled_s / jnp.maximum(counts, 1.0)
        hm = jnp.maximum(
            jnp.dot(pooled, f1w_ref[...],
                    preferred_element_type=jnp.float32) + f1b_ref[...], 0.0)
        out_ref[...] = jnp.dot(hm, f2w_ref[...],
                               preferred_element_type=jnp.float32) + f2b_ref[...]

    return pl.pallas_call(
        body,
        out_shape=jax.ShapeDtypeStruct((64, 128), jnp.float32),
    )(outs, esp, b, g, be, batch_r, fc1_w, fc1_b, fc2_w, fc2_b)


def _esum_view(esum):
    return (esum.reshape(2, 2, 10240)[:, :, :_NP]
            .reshape(2, 2, _NP, 1))


def kernel(x, edge_index, batch,
           W1, a_src1, a_dst1, b1, g1, be1,
           W2, a_src2, a_dst2, b2, g2, be2,
           W3, a_src3, a_dst3, b3, g3, be3,
           fc1_w, fc1_b, fc2_w, fc2_b):
    f32 = jnp.float32
    i32 = jnp.int32
    x_pad = jnp.concatenate([x, jnp.zeros((_NP - _N, 128), f32)], axis=0)
    loop = jnp.arange(_N, dtype=i32)
    padv = jnp.full((_EP - _ETOT,), _PN, i32)
    src = jnp.concatenate([edge_index[0].astype(i32), loop, padv])
    dst = jnp.concatenate([edge_index[1].astype(i32), loop, padv])
    edges = jnp.stack(
        [src.reshape(_CKS, 128), dst.reshape(_CKS, 128)], axis=1)
    batch_r = jnp.concatenate(
        [batch.astype(i32), jnp.full((_NP - _N,), 64, i32)]).reshape(1, _NP)

    def mkA(a_s, a_d):
        z = jnp.zeros((64,), f32)
        return jnp.stack([
            jnp.concatenate([a_s[0], z]),
            jnp.concatenate([z, a_s[1]]),
            jnp.concatenate([a_d[0], z]),
            jnp.concatenate([z, a_d[1]]),
        ])

    A1, A2, A3 = mkA(a_src1, a_dst1), mkA(a_src2, a_dst2), mkA(a_src3, a_dst3)
    r = lambda v: v.reshape(1, 128)

    xp, scal, cvec = _tc_prep1(x_pad, W1, A1)
    o1, s1 = _sc_edge_pass(xp, scal, cvec, edges)
    xp, scal, cvec = _tc_comb(o1, _esum_view(s1), r(b1), r(g1), r(be1), W2, A2)
    o2, s2 = _sc_edge_pass(xp, scal, cvec, edges)
    xp, scal, cvec = _tc_comb(o2, _esum_view(s2), r(b2), r(g2), r(be2), W3, A3)
    o3, s3 = _sc_edge_pass(xp, scal, cvec, edges)
    return _tc_final(o3, _esum_view(s3), r(b3), r(g3), r(be3), batch_r,
                     fc1_w, r(fc1_b), fc2_w, r(fc2_b))


# speedup vs baseline: 84.7738x; 1.0975x over previous
import functools

import jax
import jax.numpy as jnp
from jax import lax
from jax.experimental import pallas as pl
from jax.experimental.pallas import tpu as pltpu
from jax.experimental.pallas import tpu_sc as plsc

_N = 10000
_NP = 10112
_PN = 10000
_E = 320000
_ETOT = _E + _N
_EP = 331776
_PT = _EP // 32
_CH = 128
_NCH = _PT // _CH
_CKS = _EP // _CH
_BT = 9
_NR = _NP // 16

_SC_PARAMS = pltpu.CompilerParams(needs_layout_passes=False)


def _sc_attn_pass(scal, cvec, edges):
    mesh = plsc.VectorSubcoreMesh(core_axis_name="c", subcore_axis_name="s")

    @functools.partial(
        pl.kernel,
        out_type=(
            jax.ShapeDtypeStruct((_CKS, 2, 128), jnp.float32),
            jax.ShapeDtypeStruct((2, 160, 128), jnp.float32),
        ),
        mesh=mesh,
        compiler_params=_SC_PARAMS,
        scratch_types=[
            pltpu.VMEM_SHARED((160, 128), jnp.float32),
            pltpu.VMEM((_NP,), jnp.float32),
            pltpu.VMEM((_NP,), jnp.float32),
            pltpu.VMEM((_NP,), jnp.float32),
            pltpu.VMEM((_NP,), jnp.float32),
            pltpu.VMEM((80, 128), jnp.float32),
            pltpu.VMEM((80, 128), jnp.float32),
            pltpu.VMEM((_BT, 2, 128), jnp.int32),
            pltpu.VMEM((_BT, 2, 128), jnp.float32),
            pltpu.VMEM((80,), jnp.int32),
            pltpu.VMEM((80,), jnp.int32),
            pltpu.VMEM((2, 16), jnp.float32),
        ],
    )
    def k(scal_hbm, cvec_hbm, ed_hbm, ee_hbm, esum_hbm,
          sp, as0, as1, ad0, ad1, es0, es1, edb, eeb, eidx0, eidx1, cbuf):
        cid = lax.axis_index("c")
        sid = lax.axis_index("s")
        z16 = jnp.zeros((16,), jnp.float32)

        def zes(r, carry):
            for c in range(8):
                es0[r, pl.ds(c * 16, 16)] = z16
                es1[r, pl.ds(c * 16, 16)] = z16
            return carry

        lax.fori_loop(0, 80, zes, 0)

        @pl.when(sid < 10)
        def _():
            pltpu.sync_copy(es0.at[pl.ds(0, 16)],
                            sp.at[pl.ds(sid * 16, 16)])

        pltpu.sync_copy(scal_hbm.at[0], as0)
        pltpu.sync_copy(scal_hbm.at[1], as1)
        pltpu.sync_copy(scal_hbm.at[2], ad0)
        pltpu.sync_copy(scal_hbm.at[3], ad1)
        pltpu.sync_copy(cvec_hbm, cbuf)
        c0v = cbuf[0, :]
        c1v = cbuf[1, :]

        ii = lax.iota(jnp.int32, 16)
        for g in range(5):
            eidx0[pl.ds(g * 16, 16)] = ii + g * 16
            eidx1[pl.ds(g * 16, 16)] = ii + (80 + g * 16)

        plsc.subcore_barrier()

        ckb = (cid * 16 + sid) * _NCH

        def batch(tt, carry):
            ck = ckb + tt * _BT
            pltpu.sync_copy(ed_hbm.at[pl.ds(ck, _BT)], edb)
            for u in range(_BT):
                for j in range(8):
                    sl = pl.ds(16 * j, 16)
                    sv = edb[u, 0, sl]
                    dv = edb[u, 1, sl]
                    rr = lax.shift_right_logical(dv, 7)
                    cc = lax.bitwise_and(dv, 127)
                    e0 = (plsc.load_gather(as0, [sv])
                          + plsc.load_gather(ad0, [dv]))
                    e0 = jnp.where(e0 > 0, e0, 0.2 * e0)
                    e0 = jnp.exp(e0 - c0v)
                    e1 = (plsc.load_gather(as1, [sv])
                          + plsc.load_gather(ad1, [dv]))
                    e1 = jnp.where(e1 > 0, e1, 0.2 * e1)
                    e1 = jnp.exp(e1 - c1v)
                    eeb[u, 0, sl] = e0
                    eeb[u, 1, sl] = e1
                    plsc.addupdate_scatter(es0, [rr, cc], e0)
                    plsc.addupdate_scatter(es1, [rr, cc], e1)
            pltpu.sync_copy(eeb, ee_hbm.at[pl.ds(ck, _BT)])
            return carry

        lax.fori_loop(0, _NCH // _BT, batch, 0)

        pltpu.sync_copy(es0, sp.at[eidx0], add=True)
        pltpu.sync_copy(es1, sp.at[eidx1], add=True)

        plsc.subcore_barrier()

        @pl.when(sid < 10)
        def _():
            pltpu.sync_copy(sp.at[pl.ds(sid * 16, 16)],
                            esum_hbm.at[cid, pl.ds(sid * 16, 16)])

    return k(scal, cvec, edges)


def _sc_agg_pass(xp, ee, edges):
    mesh = plsc.VectorSubcoreMesh(core_axis_name="c", subcore_axis_name="s")

    @functools.partial(
        pl.kernel,
        out_type=jax.ShapeDtypeStruct((2, _NP, 128), jnp.float32),
        mesh=mesh,
        compiler_params=_SC_PARAMS,
        scratch_types=[
            pltpu.VMEM_SHARED((_NP, 128), jnp.float32),
            pltpu.VMEM((2, 2, 128), jnp.int32),
            pltpu.VMEM((2, 2, 128), jnp.float32),
            pltpu.VMEM((2, 2, 128), jnp.int32),
            pltpu.VMEM((2, 2, 128), jnp.float32),
            pltpu.VMEM((_CH, 128), jnp.float32),
            pltpu.VMEM((_CH, 128), jnp.float32),
            pltpu.SemaphoreType.DMA,
            pltpu.SemaphoreType.DMA,
            pltpu.SemaphoreType.DMA,
            pltpu.SemaphoreType.DMA,
        ],
    )
    def k(xp_hbm, ee_hbm, ed_hbm, out_hbm,
          sp, edAB, eeAB, edCD, eeCD, rowsA, rowsB, sga, sgb, ssa, ssb):
        cid = lax.axis_index("c")
        sid = lax.axis_index("s")
        z16 = jnp.zeros((16,), jnp.float32)

        def zrow(r, carry):
            for c in range(8):
                rowsA[r, pl.ds(c * 16, 16)] = z16
            return carry

        lax.fori_loop(0, _CH, zrow, 0)

        base_r = sid * _NR
        for i in range(4):
            pltpu.sync_copy(rowsA.at[pl.ds(0, 128)],
                            sp.at[pl.ds(base_r + i * 128, 128)])
        pltpu.sync_copy(rowsA.at[pl.ds(0, 120)],
                        sp.at[pl.ds(base_r + 512, 120)])

        plsc.subcore_barrier()

        ckb = (cid * 16 + sid) * _NCH

        def scale(rows, eeb, a):
            def body(g, carry):
                e0g = eeb[a, 0, pl.ds(g * 16, 16)]
                e1g = eeb[a, 1, pl.ds(g * 16, 16)]
                r0 = g * 16
                for t in range(16):
                    b0 = jnp.broadcast_to(e0g[t], (16,))
                    b1 = jnp.broadcast_to(e1g[t], (16,))
                    for c in range(4):
                        sl = pl.ds(c * 16, 16)
                        rows[r0 + t, sl] = rows[r0 + t, sl] * b0
                    for c in range(4, 8):
                        sl = pl.ds(c * 16, 16)
                        rows[r0 + t, sl] = rows[r0 + t, sl] * b1
                return carry

            lax.fori_loop(0, 8, body, 0)

        def quad(i, carry):
            ck0 = ckb + 4 * i
            pltpu.sync_copy(ed_hbm.at[pl.ds(ck0, 2)], edAB)
            pltpu.sync_copy(ee_hbm.at[pl.ds(ck0, 2)], eeAB)
            gA = pltpu.async_copy(xp_hbm.at[edAB.at[0, 0]], rowsA, sga)
            gB = pltpu.async_copy(xp_hbm.at[edAB.at[1, 0]], rowsB, sgb)
            pltpu.sync_copy(ed_hbm.at[pl.ds(ck0 + 2, 2)], edCD)
            pltpu.sync_copy(ee_hbm.at[pl.ds(ck0 + 2, 2)], eeCD)
            gA.wait()
            scale(rowsA, eeAB, 0)
            sA = pltpu.async_copy(rowsA, sp.at[edAB.at[0, 1]], ssa, add=True)
            gB.wait()
            scale(rowsB, eeAB, 1)
            sB = pltpu.async_copy(rowsB, sp.at[edAB.at[1, 1]], ssb, add=True)
            sA.wait()
            gC = pltpu.async_copy(xp_hbm.at[edCD.at[0, 0]], rowsA, sga)
            sB.wait()
            gD = pltpu.async_copy(xp_hbm.at[edCD.at[1, 0]], rowsB, sgb)
            gC.wait()
            scale(rowsA, eeCD, 0)
            sC = pltpu.async_copy(rowsA, sp.at[edCD.at[0, 1]], ssa, add=True)
            gD.wait()
            scale(rowsB, eeCD, 1)
            sD = pltpu.async_copy(rowsB, sp.at[edCD.at[1, 1]], ssb, add=True)
            sC.wait()
            sD.wait()
            return carry

        lax.fori_loop(0, _NCH // 4, quad, 0)

        ckl = ckb + _NCH - 1
        pltpu.sync_copy(ed_hbm.at[pl.ds(ckl, 1)], edAB.at[pl.ds(0, 1)])
        pltpu.sync_copy(ee_hbm.at[pl.ds(ckl, 1)], eeAB.at[pl.ds(0, 1)])
        pltpu.async_copy(xp_hbm.at[edAB.at[0, 0]], rowsA, sga).wait()
        scale(rowsA, eeAB, 0)
        pltpu.sync_copy(rowsA, sp.at[edAB.at[0, 1]], add=True)

        plsc.subcore_barrier()

        for i in range(4):
            pltpu.sync_copy(sp.at[pl.ds(base_r + i * 128, 128)],
                            out_hbm.at[cid, pl.ds(base_r + i * 128, 128)])
        pltpu.sync_copy(sp.at[pl.ds(base_r + 512, 120)],
                        out_hbm.at[cid, pl.ds(base_r + 512, 120)])

    return k(xp, ee, edges)


def _sc_edge_pass(xp, scal, cvec, edges):
    ee, esum = _sc_attn_pass(scal, cvec, edges)
    acc = _sc_agg_pass(xp, ee, edges)
    return acc, esum


def _prep_tail(h, W, A):
    xpn = jnp.dot(h, W, preferred_element_type=jnp.float32)
    scal = lax.dot_general(A, xpn, (((1,), (1,)), ((), ())),
                           preferred_element_type=jnp.float32)
    m = jnp.max(scal, axis=1, keepdims=True)
    cs = m[0:2] + m[2:4]
    cvec = jnp.broadcast_to(cs, (2, 16))
    return xpn, scal, cvec


def _tc_prep1(x_pad, W1, A1):
    def body(x_ref, w_ref, a_ref, xp_ref, scal_ref, cvec_ref):
        xpn, scal, cvec = _prep_tail(x_ref[...], w_ref[...], a_ref[...])
        xp_ref[...] = xpn
        scal_ref[...] = scal
        cvec_ref[...] = cvec

    return pl.pallas_call(
        body,
        out_shape=(
            jax.ShapeDtypeStruct((_NP, 128), jnp.float32),
            jax.ShapeDtypeStruct((4, _NP), jnp.float32),
            jax.ShapeDtypeStruct((2, 16), jnp.float32),
        ),
    )(x_pad, W1, A1)


def _finish_layer(o_ref, e_ref, b_ref, g_ref, be_ref):
    acc = o_ref[0] + o_ref[1]
    es0 = e_ref[0, 0] + e_ref[1, 0]
    es1 = e_ref[0, 1] + e_ref[1, 1]
    h0 = acc[:, :64] / (es0 + 1e-16)
    h1 = acc[:, 64:] / (es1 + 1e-16)
    h = jnp.concatenate([h0, h1], axis=1) + b_ref[...]
    rid = lax.broadcasted_iota(jnp.int32, (_NP, 128), 0)
    h = jnp.where(rid < _N, h, 0.0)
    mu = jnp.sum(h, axis=0, keepdims=True) / _N
    var = jnp.sum(h * h, axis=0, keepdims=True) / _N - mu * mu
    hn = g_ref[...] * (h - mu) * lax.rsqrt(var + 1e-5) + be_ref[...]
    hr = jnp.maximum(hn, 0.0)
    return jnp.where(rid < _N, hr, 0.0)


def _tc_comb(outs, esp, b, g, be, W, A):
    def body(o_ref, e_ref, b_ref, g_ref, be_ref, w_ref, a_ref,
             xp_ref, scal_ref, cvec_ref):
        hr = _finish_layer(o_ref, e_ref, b_ref, g_ref, be_ref)
        xpn, scal, cvec = _prep_tail(hr, w_ref[...], a_ref[...])
        xp_ref[...] = xpn
        scal_ref[...] = scal
        cvec_ref[...] = cvec

    return pl.pallas_call(
        body,
        out_shape=(
            jax.ShapeDtypeStruct((_NP, 128), jnp.float32),
            jax.ShapeDtypeStruct((4, _NP), jnp.float32),
            jax.ShapeDtypeStruct((2, 16), jnp.float32),
        ),
    )(outs, esp, b, g, be, W, A)


def _tc_final(outs, esp, b, g, be, batch_r, fc1_w, fc1_b, fc2_w, fc2_b):
    def body(o_ref, e_ref, b_ref, g_ref, be_ref, bt_ref,
             f1w_ref, f1b_ref, f2w_ref, f2b_ref, out_ref):
        hr = _finish_layer(o_ref, e_ref, b_ref, g_ref, be_ref)
        gid = lax.broadcasted_iota(jnp.int32, (64, _NP), 0)
        ohT = (gid == bt_ref[...]).astype(jnp.float32)
        pooled_s = lax.dot_general(ohT, hr, (((1,), (0,)), ((), ())),
                                   preferred_element_type=jnp.float32)
        ones = jnp.ones((_NP, 1), jnp.float32)
        counts = lax.dot_general(ohT, ones, (((1,), (0,)), ((), ())),
                                 preferred_element_type=jnp.float32)
        pooled = pooled_s / jnp.maximum(counts, 1.0)
        hm = jnp.maximum(
            jnp.dot(pooled, f1w_ref[...],
                    preferred_element_type=jnp.float32) + f1b_ref[...], 0.0)
        out_ref[...] = jnp.dot(hm, f2w_ref[...],
                               preferred_element_type=jnp.float32) + f2b_ref[...]

    return pl.pallas_call(
        body,
        out_shape=jax.ShapeDtypeStruct((64, 128), jnp.float32),
    )(outs, esp, b, g, be, batch_r, fc1_w, fc1_b, fc2_w, fc2_b)


def _esum_view(esum):
    return (esum.reshape(2, 2, 10240)[:, :, :_NP]
            .reshape(2, 2, _NP, 1))


def kernel(x, edge_index, batch,
           W1, a_src1, a_dst1, b1, g1, be1,
           W2, a_src2, a_dst2, b2, g2, be2,
           W3, a_src3, a_dst3, b3, g3, be3,
           fc1_w, fc1_b, fc2_w, fc2_b):
    f32 = jnp.float32
    i32 = jnp.int32
    x_pad = jnp.concatenate([x, jnp.zeros((_NP - _N, 128), f32)], axis=0)
    loop = jnp.arange(_N, dtype=i32)
    padv = jnp.full((_EP - _ETOT,), _PN, i32)
    src = jnp.concatenate([edge_index[0].astype(i32), loop, padv])
    dst = jnp.concatenate([edge_index[1].astype(i32), loop, padv])
    edges = jnp.stack(
        [src.reshape(_CKS, 128), dst.reshape(_CKS, 128)], axis=1)
    batch_r = jnp.concatenate(
        [batch.astype(i32), jnp.full((_NP - _N,), 64, i32)]).reshape(1, _NP)

    def mkA(a_s, a_d):
        z = jnp.zeros((64,), f32)
        return jnp.stack([
            jnp.concatenate([a_s[0], z]),
            jnp.concatenate([z, a_s[1]]),
            jnp.concatenate([a_d[0], z]),
            jnp.concatenate([z, a_d[1]]),
        ])

    A1, A2, A3 = mkA(a_src1, a_dst1), mkA(a_src2, a_dst2), mkA(a_src3, a_dst3)
    r = lambda v: v.reshape(1, 128)

    xp, scal, cvec = _tc_prep1(x_pad, W1, A1)
    o1, s1 = _sc_edge_pass(xp, scal, cvec, edges)
    xp, scal, cvec = _tc_comb(o1, _esum_view(s1), r(b1), r(g1), r(be1), W2, A2)
    o2, s2 = _sc_edge_pass(xp, scal, cvec, edges)
    xp, scal, cvec = _tc_comb(o2, _esum_view(s2), r(b2), r(g2), r(be2), W3, A3)
    o3, s3 = _sc_edge_pass(xp, scal, cvec, edges)
    return _tc_final(o3, _esum_view(s3), r(b3), r(g3), r(be3), batch_r,
                     fc1_w, r(fc1_b), fc2_w, r(fc2_b))
